# baseline (speedup 1.0000x reference)
import functools

import jax
import jax.numpy as jnp
from jax import lax
from jax.experimental import pallas as pl
from jax.experimental.pallas import tpu as pltpu
from jax.experimental.pallas import tpu_sc as plsc

N = 10000
E = 320000
D_IN = 128
H = 128
MLP_H = 64
G = 32

NC, NS = 2, 16
NW = NC * NS
EPW = E // NW
K = 80
NCHUNK = EPW // K
NPAD = 10240
ROWS_PT = NPAD // NS
CW = 128

BN = 400
GRID_N = N // BN

@functools.cache
def _mesh():
    return plsc.VectorSubcoreMesh(core_axis_name="c", subcore_axis_name="s",
                                  num_cores=NC, num_subcores=NS)


def _sc_agg_body(h_hbm, edges_hbm, out_hbm,
                 idx_a, idx_b, idx_c, rows_a, rows_b, rows_c,
                 stage_v, accum_sh,
                 sem_ia, sem_ib, sem_ic, sem_a, sem_b, sem_c,
                 sem_sa, sem_sb, sem_sc):
    c = lax.axis_index("c")
    s = lax.axis_index("s")
    wid = c * NS + s

    SL = [(idx_a, sem_ia, rows_a, sem_a, sem_sa),
          (idx_b, sem_ib, rows_b, sem_b, sem_sb),
          (idx_c, sem_ic, rows_c, sem_c, sem_sc)]

    def issue(i, sl):
        idx, sem_i, rows, sem_r, _ = sl
        pltpu.async_copy(edges_hbm.at[wid, i], idx, sem_i)
        pltpu.make_async_copy(edges_hbm.at[wid, i], idx, sem_i).wait()
        pltpu.async_copy(h_hbm.at[idx.at[0]], rows, sem_r)

    def start_scatter(sl):
        idx, _, rows, sem_r, sem_s = sl
        pltpu.make_async_copy(h_hbm.at[idx.at[0]], rows, sem_r).wait()
        pltpu.async_copy(rows, accum_sh.at[idx.at[1]], sem_s, add=True)

    def wait_scatter(sl):
        idx, _, rows, _, sem_s = sl
        pltpu.make_async_copy(rows, accum_sh.at[idx.at[1]], sem_s).wait()

    issue(0, SL[0])

    zero16 = jnp.zeros((16,), jnp.float32)
    for r in range(32):
        for j in range(H // 16):
            stage_v[r, pl.ds(j * 16, 16)] = zero16
    tbase = s * ROWS_PT
    for i in range(ROWS_PT // 32):
        pltpu.sync_copy(stage_v, accum_sh.at[pl.ds(tbase + i * 32, 32)])
    plsc.subcore_barrier()

    start_scatter(SL[0])
    issue(1, SL[1])
    start_scatter(SL[1])
    issue(2, SL[2])
    start_scatter(SL[2])
    wait_scatter(SL[0])
    issue(3, SL[0])

    def body(j, _):
        i = 3 * j
        start_scatter(SL[0])
        wait_scatter(SL[1])
        issue(i + 1, SL[1])
        start_scatter(SL[1])
        wait_scatter(SL[2])
        issue(i + 2, SL[2])
        start_scatter(SL[2])
        wait_scatter(SL[0])
        issue(i + 3, SL[0])
        return 0

    lax.fori_loop(1, (NCHUNK - 2) // 3, body, 0)
    start_scatter(SL[0])
    wait_scatter(SL[1])
    issue(NCHUNK - 1, SL[1])
    start_scatter(SL[1])
    wait_scatter(SL[2])
    wait_scatter(SL[0])
    wait_scatter(SL[1])
    plsc.subcore_barrier()

    for i in range(ROWS_PT // 32):
        pltpu.sync_copy(accum_sh.at[pl.ds(tbase + i * 32, 32)], stage_v)
        pltpu.sync_copy(stage_v, out_hbm.at[c, pl.ds(tbase + i * 32, 32)])


@functools.cache
def _sc_agg_kernel():
    return pl.kernel(
        _sc_agg_body,
        out_type=jax.ShapeDtypeStruct((NC, NPAD, H), jnp.float32),
        mesh=_mesh(),
        scratch_types=(
            [pltpu.VMEM((2, K), jnp.int32)] * 3
            + [pltpu.VMEM((K, H), jnp.float32)] * 3
            + [pltpu.VMEM((32, H), jnp.float32),
               pltpu.VMEM_SHARED((NPAD, H), jnp.float32)]
            + [pltpu.SemaphoreType.DMA] * 9
        ),
    )


def _sc_agg(h, src, dst):
    edges = jnp.stack([src.reshape(NW, NCHUNK, K),
                       dst.reshape(NW, NCHUNK, K)], axis=2)
    return _sc_agg_kernel()(h, edges)


def _sc_cnt_body(dst_hbm, out_hbm, dst_v, ones_v, stage_v, accum_sh):
    c = lax.axis_index("c")
    s = lax.axis_index("s")
    wid = c * NS + s

    one16 = jnp.ones((16,), jnp.float32)
    zero16 = jnp.zeros((16,), jnp.float32)
    for r in range(K):
        for j in range(CW // 16):
            ones_v[r, pl.ds(j * 16, 16)] = one16
    for r in range(32):
        for j in range(CW // 16):
            stage_v[r, pl.ds(j * 16, 16)] = zero16
    tbase = s * ROWS_PT
    for i in range(ROWS_PT // 32):
        pltpu.sync_copy(stage_v, accum_sh.at[pl.ds(tbase + i * 32, 32)])
    plsc.subcore_barrier()

    ebase = wid * EPW

    def chunk(i, _):
        off = ebase + i * K
        pltpu.sync_copy(dst_hbm.at[pl.ds(off, K)], dst_v)
        pltpu.sync_copy(ones_v, accum_sh.at[dst_v], add=True)
        return 0

    lax.fori_loop(0, NCHUNK, chunk, 0)
    plsc.subcore_barrier()

    for i in range(ROWS_PT // 32):
        pltpu.sync_copy(accum_sh.at[pl.ds(tbase + i * 32, 32)], stage_v)
        pltpu.sync_copy(stage_v, out_hbm.at[c, pl.ds(tbase + i * 32, 32)])


@functools.cache
def _sc_cnt_kernel():
    return pl.kernel(
        _sc_cnt_body,
        out_type=jax.ShapeDtypeStruct((NC, NPAD, CW), jnp.float32),
        mesh=_mesh(),
        scratch_types=[
            pltpu.VMEM((K,), jnp.int32),
            pltpu.VMEM((K, CW), jnp.float32),
            pltpu.VMEM((32, CW), jnp.float32),
            pltpu.VMEM_SHARED((NPAD, CW), jnp.float32),
        ],
    )


def _sc_cnt(dst):
    return _sc_cnt_kernel()(dst)


def _ln(z, g, b):
    mu = jnp.mean(z, axis=-1, keepdims=True)
    var = jnp.mean(jnp.square(z - mu), axis=-1, keepdims=True)
    return (z - mu) * lax.rsqrt(var + 1e-5) * g + b


def _dot(a, b):
    return jnp.dot(a, b, preferred_element_type=jnp.float32)


def _enc_body(x_ref, w0, b0, w1, b1, w2, b2, lng, lnb, w3, b3, o_ref):
    h = jnp.maximum(_dot(x_ref[...], w0[...]) + b0[...], 0.0)
    h = jnp.maximum(_dot(h, w1[...]) + b1[...], 0.0)
    h = jnp.maximum(_dot(h, w2[...]) + b2[...], 0.0)
    h = _ln(h, lng[...], lnb[...])
    o_ref[...] = _dot(h, w3[...]) + b3[...]


def _full(shape):
    return pl.BlockSpec(shape, lambda i: (0,) * len(shape))


def _encoder(x, p):
    specs = [pl.BlockSpec((BN, D_IN), lambda i: (i, 0)),
             _full((D_IN, MLP_H)), _full((1, MLP_H)),
             _full((MLP_H, MLP_H)), _full((1, MLP_H)),
             _full((MLP_H, MLP_H)), _full((1, MLP_H)),
             _full((1, MLP_H)), _full((1, MLP_H)),
             _full((MLP_H, H)), _full((1, H))]
    return pl.pallas_call(
        _enc_body,
        grid=(GRID_N,),
        in_specs=specs,
        out_specs=pl.BlockSpec((BN, H), lambda i: (i, 0)),
        out_shape=jax.ShapeDtypeStruct((N, H), jnp.float32),
    )(x, p['enc_w0'], p['enc_b0'].reshape(1, -1),
      p['enc_w1'], p['enc_b1'].reshape(1, -1),
      p['enc_w2'], p['enc_b2'].reshape(1, -1),
      p['enc_ln_g'].reshape(1, -1), p['enc_ln_b'].reshape(1, -1),
      p['enc_w3'], p['enc_b3'].reshape(1, -1))


def _conv_body(p0, p1, c0, c1, h_ref, wl, bl, wr, o_ref):
    cnt = c0[:, :1] + c1[:, :1]
    inv = 1.0 / jnp.maximum(cnt, 1.0)
    mean = (p0[...] + p1[...]) * inv
    o_ref[...] = jnp.maximum(
        _dot(mean, wl[...]) + bl[...] + _dot(h_ref[...], wr[...]), 0.0)


def _conv(parts, cnt, h, wl, bl, wr):
    specs = [pl.BlockSpec((BN, H), lambda i: (i, 0)),
             pl.BlockSpec((BN, H), lambda i: (i, 0)),
             pl.BlockSpec((BN, CW), lambda i: (i, 0)),
             pl.BlockSpec((BN, CW), lambda i: (i, 0)),
             pl.BlockSpec((BN, H), lambda i: (i, 0)),
             _full((H, H)), _full((1, H)), _full((H, H))]
    return pl.pallas_call(
        _conv_body,
        grid=(GRID_N,),
        in_specs=specs,
        out_specs=pl.BlockSpec((BN, H), lambda i: (i, 0)),
        out_shape=jax.ShapeDtypeStruct((N, H), jnp.float32),
    )(parts[0], parts[1], cnt[0], cnt[1], h, wl, bl.reshape(1, -1), wr)


def _pool_dec_body(b_ref, h_ref,
                   ln0g0, ln0b0, w00, b00, ln1g0, ln1b0, w10, b10,
                   ln0g1, ln0b1, w01, b01, ln1g1, ln1b1, w11, b11,
                   o_ref, acc_ref):
    i = pl.program_id(0)
    bv = jnp.broadcast_to(b_ref[0], (G, BN))
    ids = lax.broadcasted_iota(jnp.int32, (G, BN), 0)
    oh = jnp.where(ids == bv, 1.0, 0.0)
    part = lax.dot_general(oh, h_ref[...], (((1,), (0,)), ((), ())),
                           preferred_element_type=jnp.float32)

    @pl.when(i == 0)
    def _():
        acc_ref[...] = part

    @pl.when(i > 0)
    def _():
        acc_ref[...] = acc_ref[...] + part

    @pl.when(i == GRID_N - 1)
    def _():
        pooled = acc_ref[...]
        z0 = _ln(pooled, ln0g0[...], ln0b0[...])
        z0 = jnp.maximum(_dot(z0, w00[...]) + b00[...], 0.0)
        z0 = _ln(z0, ln1g0[...], ln1b0[...])
        z0 = jnp.maximum(_dot(z0, w10[...]) + b10[...], 0.0)
        z1 = _ln(pooled, ln0g1[...], ln0b1[...])
        z1 = jnp.maximum(_dot(z1, w01[...]) + b01[...], 0.0)
        z1 = _ln(z1, ln1g1[...], ln1b1[...])
        z1 = jnp.maximum(_dot(z1, w11[...]) + b11[...], 0.0)
        o_ref[...] = z0 + z1


def _pool_decode(batch3d, h, p):
    ins = [batch3d, h]
    for hd in range(2):
        w1 = p['dec%d_w1' % hd]
        b1 = p['dec%d_b1' % hd]
        sel = (jnp.arange(H, dtype=jnp.float32) == hd).reshape(1, H)
        ins += [p['dec%d_ln0_g' % hd].reshape(1, -1),
                p['dec%d_ln0_b' % hd].reshape(1, -1),
                p['dec%d_w0' % hd], p['dec%d_b0' % hd].reshape(1, -1),
                p['dec%d_ln1_g' % hd].reshape(1, -1),
                p['dec%d_ln1_b' % hd].reshape(1, -1),
                w1 @ sel, b1.reshape(1, 1) @ sel]
    specs = [pl.BlockSpec((1, 1, BN), lambda i: (i, 0, 0)),
             pl.BlockSpec((BN, H), lambda i: (i, 0))]
    for hd in range(2):
        specs += [_full((1, H)), _full((1, H)), _full((H, H)), _full((1, H)),
                  _full((1, H)), _full((1, H)), _full((H, H)), _full((1, H))]
    out = pl.pallas_call(
        _pool_dec_body,
        grid=(GRID_N,),
        in_specs=specs,
        out_specs=pl.BlockSpec((G, H), lambda i: (0, 0)),
        out_shape=jax.ShapeDtypeStruct((G, H), jnp.float32),
        scratch_shapes=[pltpu.VMEM((G, H), jnp.float32)],
    )(*ins)
    return out[:, :2]


def kernel(x, edge_index, batch, params):
    src = edge_index[0]
    dst = edge_index[1]
    h = _encoder(x, params)
    cnt = _sc_cnt(dst)
    for i in range(3):
        parts = _sc_agg(h, src, dst)
        h = _conv(parts, cnt, h,
                  params['conv%d_wl' % i], params['conv%d_bl' % i],
                  params['conv%d_wr' % i])
    return _pool_decode(batch.reshape(GRID_N, 1, BN), h, params)


# speedup vs baseline: 1.3870x; 1.3870x over previous
import functools

import jax
import jax.numpy as jnp
from jax import lax
from jax.experimental import pallas as pl
from jax.experimental.pallas import tpu as pltpu
from jax.experimental.pallas import tpu_sc as plsc

N = 10000
E = 320000
D_IN = 128
H = 128
MLP_H = 64
G = 32

NC, NS = 2, 16
NW = NC * NS
EPW = E // NW
K = 80
NCHUNK = EPW // K
NPAD = 10240
ROWS_PT = NPAD // NS
CW = 128

BN = 400
GRID_N = N // BN

@functools.cache
def _mesh():
    return plsc.VectorSubcoreMesh(core_axis_name="c", subcore_axis_name="s",
                                  num_cores=NC, num_subcores=NS)


def _sc_agg_body(h_hbm, edges_hbm, out_hbm,
                 idx_a, idx_b, idx_c, idx_d, rows_a, rows_b, rows_c, rows_d,
                 stage_v, accum_sh,
                 sem_ia, sem_ib, sem_ic, sem_id, sem_a, sem_b, sem_c, sem_d,
                 sem_sa, sem_sb, sem_sc, sem_sd):
    c = lax.axis_index("c")
    s = lax.axis_index("s")
    wid = c * NS + s

    SL = [(idx_a, sem_ia, rows_a, sem_a, sem_sa),
          (idx_b, sem_ib, rows_b, sem_b, sem_sb),
          (idx_c, sem_ic, rows_c, sem_c, sem_sc),
          (idx_d, sem_id, rows_d, sem_d, sem_sd)]

    def issue(i, sl):
        idx, sem_i, rows, sem_r, _ = sl
        pltpu.async_copy(edges_hbm.at[wid, i], idx, sem_i)
        pltpu.make_async_copy(edges_hbm.at[wid, i], idx, sem_i).wait()
        pltpu.async_copy(h_hbm.at[idx.at[0]], rows, sem_r)

    def start_scatter(sl):
        idx, _, rows, sem_r, sem_s = sl
        pltpu.make_async_copy(h_hbm.at[idx.at[0]], rows, sem_r).wait()
        pltpu.async_copy(rows, accum_sh.at[idx.at[1]], sem_s, add=True)

    def wait_scatter(sl):
        idx, _, rows, _, sem_s = sl
        pltpu.make_async_copy(rows, accum_sh.at[idx.at[1]], sem_s).wait()

    issue(0, SL[0])
    issue(1, SL[1])

    zero16 = jnp.zeros((16,), jnp.float32)
    for r in range(32):
        for j in range(H // 16):
            stage_v[r, pl.ds(j * 16, 16)] = zero16
    tbase = s * ROWS_PT
    for i in range(ROWS_PT // 32):
        pltpu.sync_copy(stage_v, accum_sh.at[pl.ds(tbase + i * 32, 32)])
    plsc.subcore_barrier()

    start_scatter(SL[0])
    issue(2, SL[2])
    start_scatter(SL[1])
    issue(3, SL[3])
    start_scatter(SL[2])
    wait_scatter(SL[0])
    issue(4, SL[0])
    start_scatter(SL[3])
    wait_scatter(SL[1])
    issue(5, SL[1])

    def body(j, _):
        i = 4 * j
        start_scatter(SL[0])
        wait_scatter(SL[2])
        issue(i + 2, SL[2])
        start_scatter(SL[1])
        wait_scatter(SL[3])
        issue(i + 3, SL[3])
        start_scatter(SL[2])
        wait_scatter(SL[0])
        issue(i + 4, SL[0])
        start_scatter(SL[3])
        wait_scatter(SL[1])
        issue(i + 5, SL[1])
        return 0

    lax.fori_loop(1, (NCHUNK - 5) // 4, body, 0)
    start_scatter(SL[0])
    wait_scatter(SL[2])
    issue(122, SL[2])
    start_scatter(SL[1])
    wait_scatter(SL[3])
    issue(123, SL[3])
    start_scatter(SL[2])
    wait_scatter(SL[0])
    issue(124, SL[0])
    start_scatter(SL[3])
    wait_scatter(SL[1])
    start_scatter(SL[0])
    wait_scatter(SL[2])
    wait_scatter(SL[3])
    wait_scatter(SL[0])
    plsc.subcore_barrier()

    for i in range(ROWS_PT // 32):
        pltpu.sync_copy(accum_sh.at[pl.ds(tbase + i * 32, 32)], stage_v)
        pltpu.sync_copy(stage_v, out_hbm.at[c, pl.ds(tbase + i * 32, 32)])


@functools.cache
def _sc_agg_kernel():
    return pl.kernel(
        _sc_agg_body,
        out_type=jax.ShapeDtypeStruct((NC, NPAD, H), jnp.float32),
        mesh=_mesh(),
        scratch_types=(
            [pltpu.VMEM((2, K), jnp.int32)] * 4
            + [pltpu.VMEM((K, H), jnp.float32)] * 4
            + [pltpu.VMEM((32, H), jnp.float32),
               pltpu.VMEM_SHARED((NPAD, H), jnp.float32)]
            + [pltpu.SemaphoreType.DMA] * 12
        ),
    )


def _sc_agg(h, src, dst):
    edges = jnp.stack([src.reshape(NW, NCHUNK, K),
                       dst.reshape(NW, NCHUNK, K)], axis=2)
    return _sc_agg_kernel()(h, edges)


def _sc_cnt_body(dst_hbm, out_hbm, dst_v, ones_v, stage_v, accum_sh):
    c = lax.axis_index("c")
    s = lax.axis_index("s")
    wid = c * NS + s

    one16 = jnp.ones((16,), jnp.float32)
    zero16 = jnp.zeros((16,), jnp.float32)
    for r in range(K):
        for j in range(CW // 16):
            ones_v[r, pl.ds(j * 16, 16)] = one16
    for r in range(32):
        for j in range(CW // 16):
            stage_v[r, pl.ds(j * 16, 16)] = zero16
    tbase = s * ROWS_PT
    for i in range(ROWS_PT // 32):
        pltpu.sync_copy(stage_v, accum_sh.at[pl.ds(tbase + i * 32, 32)])
    plsc.subcore_barrier()

    ebase = wid * EPW

    def chunk(i, _):
        off = ebase + i * K
        pltpu.sync_copy(dst_hbm.at[pl.ds(off, K)], dst_v)
        pltpu.sync_copy(ones_v, accum_sh.at[dst_v], add=True)
        return 0

    lax.fori_loop(0, NCHUNK, chunk, 0)
    plsc.subcore_barrier()

    for i in range(ROWS_PT // 32):
        pltpu.sync_copy(accum_sh.at[pl.ds(tbase + i * 32, 32)], stage_v)
        pltpu.sync_copy(stage_v, out_hbm.at[c, pl.ds(tbase + i * 32, 32)])


@functools.cache
def _sc_cnt_kernel():
    return pl.kernel(
        _sc_cnt_body,
        out_type=jax.ShapeDtypeStruct((NC, NPAD, CW), jnp.float32),
        mesh=_mesh(),
        scratch_types=[
            pltpu.VMEM((K,), jnp.int32),
            pltpu.VMEM((K, CW), jnp.float32),
            pltpu.VMEM((32, CW), jnp.float32),
            pltpu.VMEM_SHARED((NPAD, CW), jnp.float32),
        ],
    )


def _sc_cnt(dst):
    return _sc_cnt_kernel()(dst)


def _ln(z, g, b):
    mu = jnp.mean(z, axis=-1, keepdims=True)
    var = jnp.mean(jnp.square(z - mu), axis=-1, keepdims=True)
    return (z - mu) * lax.rsqrt(var + 1e-5) * g + b


def _dot(a, b):
    return jnp.dot(a, b, preferred_element_type=jnp.float32)


def _enc_body(x_ref, w0, b0, w1, b1, w2, b2, lng, lnb, w3, b3, o_ref):
    h = jnp.maximum(_dot(x_ref[...], w0[...]) + b0[...], 0.0)
    h = jnp.maximum(_dot(h, w1[...]) + b1[...], 0.0)
    h = jnp.maximum(_dot(h, w2[...]) + b2[...], 0.0)
    h = _ln(h, lng[...], lnb[...])
    o_ref[...] = _dot(h, w3[...]) + b3[...]


def _full(shape):
    return pl.BlockSpec(shape, lambda i: (0,) * len(shape))


def _encoder(x, p):
    specs = [pl.BlockSpec((BN, D_IN), lambda i: (i, 0)),
             _full((D_IN, MLP_H)), _full((1, MLP_H)),
             _full((MLP_H, MLP_H)), _full((1, MLP_H)),
             _full((MLP_H, MLP_H)), _full((1, MLP_H)),
             _full((1, MLP_H)), _full((1, MLP_H)),
             _full((MLP_H, H)), _full((1, H))]
    return pl.pallas_call(
        _enc_body,
        grid=(GRID_N,),
        in_specs=specs,
        out_specs=pl.BlockSpec((BN, H), lambda i: (i, 0)),
        out_shape=jax.ShapeDtypeStruct((N, H), jnp.float32),
    )(x, p['enc_w0'], p['enc_b0'].reshape(1, -1),
      p['enc_w1'], p['enc_b1'].reshape(1, -1),
      p['enc_w2'], p['enc_b2'].reshape(1, -1),
      p['enc_ln_g'].reshape(1, -1), p['enc_ln_b'].reshape(1, -1),
      p['enc_w3'], p['enc_b3'].reshape(1, -1))


def _conv_body(p0, p1, c0, c1, h_ref, wl, bl, wr, o_ref):
    cnt = c0[:, :1] + c1[:, :1]
    inv = 1.0 / jnp.maximum(cnt, 1.0)
    mean = (p0[...] + p1[...]) * inv
    o_ref[...] = jnp.maximum(
        _dot(mean, wl[...]) + bl[...] + _dot(h_ref[...], wr[...]), 0.0)


def _conv(parts, cnt, h, wl, bl, wr):
    specs = [pl.BlockSpec((BN, H), lambda i: (i, 0)),
             pl.BlockSpec((BN, H), lambda i: (i, 0)),
             pl.BlockSpec((BN, CW), lambda i: (i, 0)),
             pl.BlockSpec((BN, CW), lambda i: (i, 0)),
             pl.BlockSpec((BN, H), lambda i: (i, 0)),
             _full((H, H)), _full((1, H)), _full((H, H))]
    return pl.pallas_call(
        _conv_body,
        grid=(GRID_N,),
        in_specs=specs,
        out_specs=pl.BlockSpec((BN, H), lambda i: (i, 0)),
        out_shape=jax.ShapeDtypeStruct((N, H), jnp.float32),
    )(parts[0], parts[1], cnt[0], cnt[1], h, wl, bl.reshape(1, -1), wr)


def _pool_dec_body(b_ref, h_ref,
                   ln0g0, ln0b0, w00, b00, ln1g0, ln1b0, w10, b10,
                   ln0g1, ln0b1, w01, b01, ln1g1, ln1b1, w11, b11,
                   o_ref, acc_ref):
    i = pl.program_id(0)
    bv = jnp.broadcast_to(b_ref[0], (G, BN))
    ids = lax.broadcasted_iota(jnp.int32, (G, BN), 0)
    oh = jnp.where(ids == bv, 1.0, 0.0)
    part = lax.dot_general(oh, h_ref[...], (((1,), (0,)), ((), ())),
                           preferred_element_type=jnp.float32)

    @pl.when(i == 0)
    def _():
        acc_ref[...] = part

    @pl.when(i > 0)
    def _():
        acc_ref[...] = acc_ref[...] + part

    @pl.when(i == GRID_N - 1)
    def _():
        pooled = acc_ref[...]
        z0 = _ln(pooled, ln0g0[...], ln0b0[...])
        z0 = jnp.maximum(_dot(z0, w00[...]) + b00[...], 0.0)
        z0 = _ln(z0, ln1g0[...], ln1b0[...])
        z0 = jnp.maximum(_dot(z0, w10[...]) + b10[...], 0.0)
        z1 = _ln(pooled, ln0g1[...], ln0b1[...])
        z1 = jnp.maximum(_dot(z1, w01[...]) + b01[...], 0.0)
        z1 = _ln(z1, ln1g1[...], ln1b1[...])
        z1 = jnp.maximum(_dot(z1, w11[...]) + b11[...], 0.0)
        o_ref[...] = z0 + z1


def _pool_decode(batch3d, h, p):
    ins = [batch3d, h]
    for hd in range(2):
        w1 = p['dec%d_w1' % hd]
        b1 = p['dec%d_b1' % hd]
        sel = (jnp.arange(H, dtype=jnp.float32) == hd).reshape(1, H)
        ins += [p['dec%d_ln0_g' % hd].reshape(1, -1),
                p['dec%d_ln0_b' % hd].reshape(1, -1),
                p['dec%d_w0' % hd], p['dec%d_b0' % hd].reshape(1, -1),
                p['dec%d_ln1_g' % hd].reshape(1, -1),
                p['dec%d_ln1_b' % hd].reshape(1, -1),
                w1 @ sel, b1.reshape(1, 1) @ sel]
    specs = [pl.BlockSpec((1, 1, BN), lambda i: (i, 0, 0)),
             pl.BlockSpec((BN, H), lambda i: (i, 0))]
    for hd in range(2):
        specs += [_full((1, H)), _full((1, H)), _full((H, H)), _full((1, H)),
                  _full((1, H)), _full((1, H)), _full((H, H)), _full((1, H))]
    out = pl.pallas_call(
        _pool_dec_body,
        grid=(GRID_N,),
        in_specs=specs,
        out_specs=pl.BlockSpec((G, H), lambda i: (0, 0)),
        out_shape=jax.ShapeDtypeStruct((G, H), jnp.float32),
        scratch_shapes=[pltpu.VMEM((G, H), jnp.float32)],
    )(*ins)
    return out[:, :2]


def kernel(x, edge_index, batch, params):
    src = edge_index[0]
    dst = edge_index[1]
    h = _encoder(x, params)
    cnt = _sc_cnt(dst)
    for i in range(3):
        parts = _sc_agg(h, src, dst)
        h = _conv(parts, cnt, h,
                  params['conv%d_wl' % i], params['conv%d_bl' % i],
                  params['conv%d_wr' % i])
    return _pool_decode(batch.reshape(GRID_N, 1, BN), h, params)


# speedup vs baseline: 1.5357x; 1.1072x over previous
import functools

import jax
import jax.numpy as jnp
from jax import lax
from jax.experimental import pallas as pl
from jax.experimental.pallas import tpu as pltpu
from jax.experimental.pallas import tpu_sc as plsc

N = 10000
E = 320000
D_IN = 128
H = 128
MLP_H = 64
G = 32

NC, NS = 2, 16
NW = NC * NS
EPW = E // NW
K = 80
NCHUNK = EPW // K
NPAD = 10240
ROWS_PT = NPAD // NS
CW = 128

BN = 400
GRID_N = N // BN

@functools.cache
def _mesh():
    return plsc.VectorSubcoreMesh(core_axis_name="c", subcore_axis_name="s",
                                  num_cores=NC, num_subcores=NS)


def _sc_agg_body(h_hbm, edges_hbm, out_hbm,
                 idx_a, idx_b, idx_c, idx_d, rows_a, rows_b, rows_c, rows_d,
                 stage_v, accum_sh,
                 sem_ia, sem_ib, sem_ic, sem_id, sem_a, sem_b, sem_c, sem_d,
                 sem_sa, sem_sb, sem_sc, sem_sd, sem_z):
    c = lax.axis_index("c")
    s = lax.axis_index("s")
    wid = c * NS + s

    SL = [(idx_a, sem_ia, rows_a, sem_a, sem_sa),
          (idx_b, sem_ib, rows_b, sem_b, sem_sb),
          (idx_c, sem_ic, rows_c, sem_c, sem_sc),
          (idx_d, sem_id, rows_d, sem_d, sem_sd)]

    def issue(i, sl):
        idx, sem_i, rows, sem_r, _ = sl
        pltpu.async_copy(edges_hbm.at[wid, i], idx, sem_i)
        pltpu.make_async_copy(edges_hbm.at[wid, i], idx, sem_i).wait()
        pltpu.async_copy(h_hbm.at[idx.at[0]], rows, sem_r)

    def start_scatter(sl):
        idx, _, rows, sem_r, sem_s = sl
        pltpu.make_async_copy(h_hbm.at[idx.at[0]], rows, sem_r).wait()
        pltpu.async_copy(rows, accum_sh.at[idx.at[1]], sem_s, add=True)

    def wait_scatter(sl):
        idx, _, rows, _, sem_s = sl
        pltpu.make_async_copy(rows, accum_sh.at[idx.at[1]], sem_s).wait()

    issue(0, SL[0])
    issue(1, SL[1])

    zero16 = jnp.zeros((16,), jnp.float32)
    for r in range(32):
        for j in range(H // 16):
            stage_v[r, pl.ds(j * 16, 16)] = zero16
    tbase = s * ROWS_PT
    for i in range(ROWS_PT // 32):
        pltpu.async_copy(stage_v, accum_sh.at[pl.ds(tbase + i * 32, 32)],
                         sem_z)
    for i in range(ROWS_PT // 32):
        pltpu.make_async_copy(stage_v, accum_sh.at[pl.ds(tbase + i * 32, 32)],
                              sem_z).wait()
    plsc.subcore_barrier()

    start_scatter(SL[0])
    issue(2, SL[2])
    start_scatter(SL[1])
    issue(3, SL[3])
    start_scatter(SL[2])
    wait_scatter(SL[0])
    issue(4, SL[0])
    start_scatter(SL[3])
    wait_scatter(SL[1])
    issue(5, SL[1])

    def body(j, _):
        i = 4 * j
        start_scatter(SL[0])
        wait_scatter(SL[2])
        issue(i + 2, SL[2])
        start_scatter(SL[1])
        wait_scatter(SL[3])
        issue(i + 3, SL[3])
        start_scatter(SL[2])
        wait_scatter(SL[0])
        issue(i + 4, SL[0])
        start_scatter(SL[3])
        wait_scatter(SL[1])
        issue(i + 5, SL[1])
        return 0

    lax.fori_loop(1, (NCHUNK - 5) // 4, body, 0)
    start_scatter(SL[0])
    wait_scatter(SL[2])
    issue(122, SL[2])
    start_scatter(SL[1])
    wait_scatter(SL[3])
    issue(123, SL[3])
    start_scatter(SL[2])
    wait_scatter(SL[0])
    issue(124, SL[0])
    start_scatter(SL[3])
    wait_scatter(SL[1])
    start_scatter(SL[0])
    wait_scatter(SL[2])
    wait_scatter(SL[3])
    wait_scatter(SL[0])
    plsc.subcore_barrier()

    NWB = ROWS_PT // K
    for i in range(NWB):
        _, _, rows, sem_r, sem_s = SL[i % 4]
        rb = pl.ds(tbase + i * K, K)
        if i >= 4:
            pltpu.make_async_copy(rows, out_hbm.at[c, pl.ds(0, K)],
                                  sem_s).wait()
        pltpu.async_copy(accum_sh.at[rb], rows, sem_r)
        pltpu.make_async_copy(accum_sh.at[rb], rows, sem_r).wait()
        pltpu.async_copy(rows, out_hbm.at[c, rb], sem_s)
    for i in range(4):
        _, _, rows, _, sem_s = SL[i]
        pltpu.make_async_copy(rows, out_hbm.at[c, pl.ds(0, K)], sem_s).wait()


@functools.cache
def _sc_agg_kernel():
    return pl.kernel(
        _sc_agg_body,
        out_type=jax.ShapeDtypeStruct((NC, NPAD, H), jnp.float32),
        mesh=_mesh(),
        scratch_types=(
            [pltpu.VMEM((2, K), jnp.int32)] * 4
            + [pltpu.VMEM((K, H), jnp.float32)] * 4
            + [pltpu.VMEM((32, H), jnp.float32),
               pltpu.VMEM_SHARED((NPAD, H), jnp.float32)]
            + [pltpu.SemaphoreType.DMA] * 13
        ),
    )


def _sc_agg(h, edges):
    return _sc_agg_kernel()(h, edges)


def _sc_cnt_body(edges_hbm, out_hbm,
                 idx_a, idx_b, idx_c, idx_d, ones_v, wb_a, wb_b, stage_v,
                 accum_sh,
                 sem_ia, sem_ib, sem_ic, sem_id,
                 sem_sa, sem_sb, sem_sc, sem_sd, sem_z, sem_wa, sem_wb):
    c = lax.axis_index("c")
    s = lax.axis_index("s")
    wid = c * NS + s
    ISL = [(idx_a, sem_ia, sem_sa), (idx_b, sem_ib, sem_sb),
           (idx_c, sem_ic, sem_sc), (idx_d, sem_id, sem_sd)]

    def issue_idx(i, sl):
        idx, sem_i, _ = sl
        pltpu.async_copy(edges_hbm.at[wid, i], idx, sem_i)

    def start_scatter(i, sl):
        idx, sem_i, sem_s = sl
        pltpu.make_async_copy(edges_hbm.at[wid, i], idx, sem_i).wait()
        pltpu.async_copy(ones_v, accum_sh.at[idx.at[1]], sem_s, add=True)

    def wait_scatter(sl):
        idx, _, sem_s = sl
        pltpu.make_async_copy(ones_v, accum_sh.at[idx.at[1]], sem_s).wait()

    for b in range(4):
        issue_idx(b, ISL[b])

    one16 = jnp.ones((16,), jnp.float32)
    zero16 = jnp.zeros((16,), jnp.float32)
    for r in range(K):
        for j in range(CW // 16):
            ones_v[r, pl.ds(j * 16, 16)] = one16
    for r in range(32):
        for j in range(CW // 16):
            stage_v[r, pl.ds(j * 16, 16)] = zero16
    tbase = s * ROWS_PT
    for i in range(ROWS_PT // 32):
        pltpu.async_copy(stage_v, accum_sh.at[pl.ds(tbase + i * 32, 32)],
                         sem_z)
    for i in range(ROWS_PT // 32):
        pltpu.make_async_copy(stage_v, accum_sh.at[pl.ds(tbase + i * 32, 32)],
                              sem_z).wait()
    plsc.subcore_barrier()

    for b in range(4):
        start_scatter(b, ISL[b])

    def step4(j, _):
        i = 4 * j
        for b in range(4):
            sl = ISL[b]
            wait_scatter(sl)
            issue_idx(i + 4 + b, sl)
            start_scatter(i + 4 + b, sl)
        return 0

    lax.fori_loop(0, (NCHUNK - 4) // 4, step4, 0)
    wait_scatter(ISL[0])
    issue_idx(NCHUNK - 1, ISL[0])
    start_scatter(NCHUNK - 1, ISL[0])
    for sl in ISL:
        wait_scatter(sl)
    plsc.subcore_barrier()

    WB = [(wb_a, sem_wa), (wb_b, sem_wb)]
    for i in range(ROWS_PT // K):
        rows, sem_w = WB[i % 2]
        rb = pl.ds(tbase + i * K, K)
        if i >= 2:
            pltpu.make_async_copy(rows, out_hbm.at[c, pl.ds(0, K)],
                                  sem_w).wait()
        pltpu.async_copy(accum_sh.at[rb], rows, sem_w)
        pltpu.make_async_copy(accum_sh.at[rb], rows, sem_w).wait()
        pltpu.async_copy(rows, out_hbm.at[c, rb], sem_w)
    for rows, sem_w in WB:
        pltpu.make_async_copy(rows, out_hbm.at[c, pl.ds(0, K)], sem_w).wait()


@functools.cache
def _sc_cnt_kernel():
    return pl.kernel(
        _sc_cnt_body,
        out_type=jax.ShapeDtypeStruct((NC, NPAD, CW), jnp.float32),
        mesh=_mesh(),
        scratch_types=(
            [pltpu.VMEM((2, K), jnp.int32)] * 4
            + [pltpu.VMEM((K, CW), jnp.float32)] * 3
            + [pltpu.VMEM((32, CW), jnp.float32),
               pltpu.VMEM_SHARED((NPAD, CW), jnp.float32)]
            + [pltpu.SemaphoreType.DMA] * 11
        ),
    )


def _sc_cnt(edges):
    return _sc_cnt_kernel()(edges)


def _ln(z, g, b):
    mu = jnp.mean(z, axis=-1, keepdims=True)
    var = jnp.mean(jnp.square(z - mu), axis=-1, keepdims=True)
    return (z - mu) * lax.rsqrt(var + 1e-5) * g + b


def _dot(a, b):
    return jnp.dot(a, b, preferred_element_type=jnp.float32)


def _enc_body(x_ref, w0, b0, w1, b1, w2, b2, lng, lnb, w3, b3, o_ref):
    h = jnp.maximum(_dot(x_ref[...], w0[...]) + b0[...], 0.0)
    h = jnp.maximum(_dot(h, w1[...]) + b1[...], 0.0)
    h = jnp.maximum(_dot(h, w2[...]) + b2[...], 0.0)
    h = _ln(h, lng[...], lnb[...])
    o_ref[...] = _dot(h, w3[...]) + b3[...]


def _full(shape):
    return pl.BlockSpec(shape, lambda i: (0,) * len(shape))


def _encoder(x, p):
    specs = [pl.BlockSpec((BN, D_IN), lambda i: (i, 0)),
             _full((D_IN, MLP_H)), _full((1, MLP_H)),
             _full((MLP_H, MLP_H)), _full((1, MLP_H)),
             _full((MLP_H, MLP_H)), _full((1, MLP_H)),
             _full((1, MLP_H)), _full((1, MLP_H)),
             _full((MLP_H, H)), _full((1, H))]
    return pl.pallas_call(
        _enc_body,
        grid=(GRID_N,),
        in_specs=specs,
        out_specs=pl.BlockSpec((BN, H), lambda i: (i, 0)),
        out_shape=jax.ShapeDtypeStruct((N, H), jnp.float32),
    )(x, p['enc_w0'], p['enc_b0'].reshape(1, -1),
      p['enc_w1'], p['enc_b1'].reshape(1, -1),
      p['enc_w2'], p['enc_b2'].reshape(1, -1),
      p['enc_ln_g'].reshape(1, -1), p['enc_ln_b'].reshape(1, -1),
      p['enc_w3'], p['enc_b3'].reshape(1, -1))


def _conv_body(p0, p1, c0, c1, h_ref, wl, bl, wr, o_ref):
    cnt = c0[:, :1] + c1[:, :1]
    inv = 1.0 / jnp.maximum(cnt, 1.0)
    mean = (p0[...] + p1[...]) * inv
    o_ref[...] = jnp.maximum(
        _dot(mean, wl[...]) + bl[...] + _dot(h_ref[...], wr[...]), 0.0)


def _conv(parts, cnt, h, wl, bl, wr):
    specs = [pl.BlockSpec((BN, H), lambda i: (i, 0)),
             pl.BlockSpec((BN, H), lambda i: (i, 0)),
             pl.BlockSpec((BN, CW), lambda i: (i, 0)),
             pl.BlockSpec((BN, CW), lambda i: (i, 0)),
             pl.BlockSpec((BN, H), lambda i: (i, 0)),
             _full((H, H)), _full((1, H)), _full((H, H))]
    return pl.pallas_call(
        _conv_body,
        grid=(GRID_N,),
        in_specs=specs,
        out_specs=pl.BlockSpec((BN, H), lambda i: (i, 0)),
        out_shape=jax.ShapeDtypeStruct((N, H), jnp.float32),
    )(parts[0], parts[1], cnt[0], cnt[1], h, wl, bl.reshape(1, -1), wr)


def _pool_dec_body(b_ref, h_ref,
                   ln0g0, ln0b0, w00, b00, ln1g0, ln1b0, w10, b10,
                   ln0g1, ln0b1, w01, b01, ln1g1, ln1b1, w11, b11,
                   o_ref, acc_ref):
    i = pl.program_id(0)
    bv = jnp.broadcast_to(b_ref[0], (G, BN))
    ids = lax.broadcasted_iota(jnp.int32, (G, BN), 0)
    oh = jnp.where(ids == bv, 1.0, 0.0)
    part = lax.dot_general(oh, h_ref[...], (((1,), (0,)), ((), ())),
                           preferred_element_type=jnp.float32)

    @pl.when(i == 0)
    def _():
        acc_ref[...] = part

    @pl.when(i > 0)
    def _():
        acc_ref[...] = acc_ref[...] + part

    @pl.when(i == GRID_N - 1)
    def _():
        pooled = acc_ref[...]
        z0 = _ln(pooled, ln0g0[...], ln0b0[...])
        z0 = jnp.maximum(_dot(z0, w00[...]) + b00[...], 0.0)
        z0 = _ln(z0, ln1g0[...], ln1b0[...])
        z0 = jnp.maximum(_dot(z0, w10[...]) + b10[...], 0.0)
        z1 = _ln(pooled, ln0g1[...], ln0b1[...])
        z1 = jnp.maximum(_dot(z1, w01[...]) + b01[...], 0.0)
        z1 = _ln(z1, ln1g1[...], ln1b1[...])
        z1 = jnp.maximum(_dot(z1, w11[...]) + b11[...], 0.0)
        o_ref[...] = z0 + z1


def _pool_decode(batch3d, h, p):
    ins = [batch3d, h]
    for hd in range(2):
        w1 = p['dec%d_w1' % hd]
        b1 = p['dec%d_b1' % hd]
        sel = (jnp.arange(H, dtype=jnp.float32) == hd).reshape(1, H)
        ins += [p['dec%d_ln0_g' % hd].reshape(1, -1),
                p['dec%d_ln0_b' % hd].reshape(1, -1),
                p['dec%d_w0' % hd], p['dec%d_b0' % hd].reshape(1, -1),
                p['dec%d_ln1_g' % hd].reshape(1, -1),
                p['dec%d_ln1_b' % hd].reshape(1, -1),
                w1 @ sel, b1.reshape(1, 1) @ sel]
    specs = [pl.BlockSpec((1, 1, BN), lambda i: (i, 0, 0)),
             pl.BlockSpec((BN, H), lambda i: (i, 0))]
    for hd in range(2):
        specs += [_full((1, H)), _full((1, H)), _full((H, H)), _full((1, H)),
                  _full((1, H)), _full((1, H)), _full((H, H)), _full((1, H))]
    out = pl.pallas_call(
        _pool_dec_body,
        grid=(GRID_N,),
        in_specs=specs,
        out_specs=pl.BlockSpec((G, H), lambda i: (0, 0)),
        out_shape=jax.ShapeDtypeStruct((G, H), jnp.float32),
        scratch_shapes=[pltpu.VMEM((G, H), jnp.float32)],
    )(*ins)
    return out[:, :2]


def kernel(x, edge_index, batch, params):
    src = edge_index[0]
    dst = edge_index[1]
    edges = jnp.stack([src.reshape(NW, NCHUNK, K),
                       dst.reshape(NW, NCHUNK, K)], axis=2)
    h = _encoder(x, params)
    cnt = _sc_cnt(edges)
    for i in range(3):
        parts = _sc_agg(h, edges)
        h = _conv(parts, cnt, h,
                  params['conv%d_wl' % i], params['conv%d_bl' % i],
                  params['conv%d_wr' % i])
    return _pool_decode(batch.reshape(GRID_N, 1, BN), h, params)
